# Initial kernel scaffold; baseline (speedup 1.0000x reference)
import jax
import jax.numpy as jnp
from jax.experimental import pallas as pl


def kernel(x, edge_index, batch, morgan, maccs, gin_W1, gin_b1, gin_bn1_g, gin_bn1_b, gin_W2, gin_b2, gin_eps, bn_g, bn_b, vn_emb, vn_W1, vn_b1, vn_bn1_g, vn_bn1_b, vn_W2, vn_b2, vn_bn2_g, vn_bn2_b, pred_W1, pred_b1, pred_W2, pred_b2):
    raise NotImplementedError("write your pallas kernel here")


# speedup vs baseline: 4.5312x; 4.5312x over previous
import functools

import jax
import jax.numpy as jnp
from jax import lax
from jax.experimental import pallas as pl
from jax.experimental.pallas import tpu as pltpu
from jax.experimental.pallas import tpu_sc as plsc

N = 10000
E = 320000
D = 128
G = 128
NUM_TASK = 10

NC = 2
NS = 16
NW = NC * NS
CHUNK = 128
CHUNKS_PER_CORE = (E // CHUNK) // NC
FULL_TILE_CHUNKS = CHUNKS_PER_CORE // NS
ROWS_PER_TILE = 632
SEG = 312
NEG = -3.4028235e38


def _edge_agg_body(h_hbm, src_hbm, dst_hbm, out_hbm,
                   src_v, dst_v, rows_v, acc_sh, sem):
    c = lax.axis_index("c")
    s = lax.axis_index("s")

    def _zrow(i, carry):
        for r in range(8):
            rows_v[i, pl.ds(r * 16, 16)] = jnp.zeros((16,), jnp.float32)
        return carry
    lax.fori_loop(0, CHUNK, _zrow, 0)
    base0 = s * ROWS_PER_TILE

    def _zcopy(k, carry):
        pltpu.sync_copy(rows_v, acc_sh.at[pl.ds(base0 + k * CHUNK, CHUNK)])
        return carry
    lax.fori_loop(0, 4, _zcopy, 0)

    @pl.when(s < NS - 1)
    def _():
        pltpu.sync_copy(rows_v.at[pl.ds(0, 120)],
                        acc_sh.at[pl.ds(base0 + 512, 120)])

    @pl.when(s == NS - 1)
    def _():
        pltpu.sync_copy(rows_v.at[pl.ds(0, 8)],
                        acc_sh.at[pl.ds(base0 + 512, 8)])

    plsc.subcore_barrier()

    cnt = FULL_TILE_CHUNKS + jnp.where(s < 2, 1, 0)

    def _edge_chunk(i, carry):
        chunk = c * CHUNKS_PER_CORE + s + NS * i
        base = chunk * CHUNK
        pltpu.sync_copy(src_hbm.at[pl.ds(base, CHUNK)], src_v)
        pltpu.async_copy(h_hbm.at[src_v], rows_v, sem).wait()
        pltpu.sync_copy(dst_hbm.at[pl.ds(base, CHUNK)], dst_v)
        pltpu.sync_copy(rows_v, acc_sh.at[dst_v], add=True)
        return carry
    lax.fori_loop(0, cnt, _edge_chunk, 0)

    plsc.subcore_barrier()

    def _drain(k, carry):
        b = base0 + k * CHUNK
        pltpu.sync_copy(acc_sh.at[pl.ds(b, CHUNK)], rows_v)
        pltpu.sync_copy(rows_v, out_hbm.at[c, pl.ds(b, CHUNK)])
        return carry
    lax.fori_loop(0, 4, _drain, 0)

    @pl.when(s < NS - 1)
    def _():
        pltpu.sync_copy(acc_sh.at[pl.ds(base0 + 512, 120)],
                        rows_v.at[pl.ds(0, 120)])
        pltpu.sync_copy(rows_v.at[pl.ds(0, 120)],
                        out_hbm.at[c, pl.ds(base0 + 512, 120)])

    @pl.when(s == NS - 1)
    def _():
        pltpu.sync_copy(acc_sh.at[pl.ds(base0 + 512, 8)],
                        rows_v.at[pl.ds(0, 8)])
        pltpu.sync_copy(rows_v.at[pl.ds(0, 8)],
                        out_hbm.at[c, pl.ds(base0 + 512, 8)])


@functools.cache
def _edge_agg_kernel():
    mesh = plsc.VectorSubcoreMesh(core_axis_name="c", subcore_axis_name="s")
    return pl.kernel(
        _edge_agg_body,
        out_type=jax.ShapeDtypeStruct((NC, N, D), jnp.float32),
        mesh=mesh,
        scratch_types=[
            pltpu.VMEM((CHUNK,), jnp.int32),
            pltpu.VMEM((CHUNK,), jnp.int32),
            pltpu.VMEM((CHUNK, D), jnp.float32),
            pltpu.VMEM_SHARED((N, D), jnp.float32),
            pltpu.SemaphoreType.DMA,
        ],
    )


def _edge_agg(h, src, dst):
    return _edge_agg_kernel()(h, src, dst)


def _segmax_body(h_hbm, batch_hbm, out_hbm, batch_v, rows_v, acc_v):
    c = lax.axis_index("c")
    s = lax.axis_index("s")
    wid = c * NS + s
    base = wid * SEG

    pltpu.sync_copy(batch_hbm.at[pl.ds(base, SEG)], batch_v.at[pl.ds(0, SEG)])
    pltpu.sync_copy(h_hbm.at[pl.ds(base, SEG)], rows_v.at[pl.ds(0, SEG)])

    @pl.when(wid == NW - 1)
    def _():
        pltpu.sync_copy(batch_hbm.at[pl.ds(N - 16, 16)],
                        batch_v.at[pl.ds(SEG, 16)])
        pltpu.sync_copy(h_hbm.at[pl.ds(N - 16, 16)],
                        rows_v.at[pl.ds(SEG, 16)])

    neg = jnp.full((16,), NEG, jnp.float32)

    def _irow(i, carry):
        for r in range(8):
            acc_v[i, pl.ds(r * 16, 16)] = neg
        return carry
    lax.fori_loop(0, G, _irow, 0)

    n = SEG + jnp.where(wid == NW - 1, 16, 0)

    def _node(i, carry):
        g = batch_v[pl.ds(i, 16)][0]
        for r in range(8):
            sl = pl.ds(r * 16, 16)
            acc_v[g, sl] = jnp.maximum(acc_v[g, sl], rows_v[i, sl])
        return carry
    lax.fori_loop(0, n, _node, 0)

    pltpu.sync_copy(acc_v, out_hbm.at[wid])


@functools.cache
def _segmax_kernel():
    mesh = plsc.VectorSubcoreMesh(core_axis_name="c", subcore_axis_name="s")
    return pl.kernel(
        _segmax_body,
        out_type=jax.ShapeDtypeStruct((NW, G, D), jnp.float32),
        mesh=mesh,
        scratch_types=[
            pltpu.VMEM((SEG + 16 + 16,), jnp.int32),
            pltpu.VMEM((SEG + 16, D), jnp.float32),
            pltpu.VMEM((G, D), jnp.float32),
        ],
    )


def _segmax(h, batch):
    return _segmax_kernel()(h, batch)


RB = 1000
NBLK = N // RB

_DOT = dict(preferred_element_type=jnp.float32, precision=lax.Precision.HIGHEST)


def _gin_body(*refs, last):
    if last:
        h_ref, hp_ref, agg_ref, b2d_ref, w1_ref, a1_ref, w2_ref, a2_ref, out_ref = refs
    else:
        (h_ref, hp_ref, agg_ref, b2d_ref, w1_ref, a1_ref, w2_ref, a2_ref,
         out_ref, pool_ref) = refs
    h = h_ref[...]
    hp = hp_ref[...]
    z0 = a2_ref[3:4, :] * h + agg_ref[0] + agg_ref[1]
    t = jnp.dot(z0, w1_ref[...], **_DOT)
    t = a1_ref[1:2, :] * (t + a1_ref[0:1, :]) + a1_ref[2:3, :]
    t = jnp.maximum(t, 0.0)
    u = jnp.dot(t, w2_ref[...], **_DOT)
    u = a2_ref[1:2, :] * (u + a2_ref[0:1, :]) + a2_ref[2:3, :]
    if not last:
        u = jnp.maximum(u, 0.0)
    out_ref[...] = u + hp
    if not last:
        i = pl.program_id(0)
        mask = (b2d_ref[...] ==
                lax.broadcasted_iota(jnp.int32, (RB, G), 1)).astype(jnp.float32)
        p = lax.dot_general(mask, hp, (((0,), (0,)), ((), ())), **_DOT)

        @pl.when(i == 0)
        def _():
            pool_ref[...] = p

        @pl.when(i > 0)
        def _():
            pool_ref[...] += p


def _run_gin(h, h_prev, agg, batch2d, w1, a1, w2, a2, last):
    body = functools.partial(_gin_body, last=last)
    out_shape = [jax.ShapeDtypeStruct((N, D), jnp.float32)]
    out_specs = [pl.BlockSpec((RB, D), lambda i: (i, 0))]
    if not last:
        out_shape.append(jax.ShapeDtypeStruct((G, D), jnp.float32))
        out_specs.append(pl.BlockSpec((G, D), lambda i: (0, 0)))
    in_specs = [
        pl.BlockSpec((RB, D), lambda i: (i, 0)),
        pl.BlockSpec((RB, D), lambda i: (i, 0)),
        pl.BlockSpec((NC, RB, D), lambda i: (0, i, 0)),
        pl.BlockSpec((RB, 1), lambda i: (i, 0)),
        pl.BlockSpec((D, 2 * D), lambda i: (0, 0)),
        pl.BlockSpec((8, 2 * D), lambda i: (0, 0)),
        pl.BlockSpec((2 * D, D), lambda i: (0, 0)),
        pl.BlockSpec((8, D), lambda i: (0, 0)),
    ]
    return pl.pallas_call(
        body, grid=(NBLK,), in_specs=in_specs, out_specs=out_specs,
        out_shape=out_shape,
    )(h, h_prev, agg, batch2d, w1, a1, w2, a2)


def _vn_body(pool_ref, vn_ref, w1_ref, a1_ref, w2_ref, a2_ref,
             hn_ref, b2d_ref, hnext_ref, vnout_ref):
    vt = pool_ref[...] + vn_ref[...]
    t = jnp.dot(vt, w1_ref[...], **_DOT)
    t = a1_ref[1:2, :] * (t + a1_ref[0:1, :]) + a1_ref[2:3, :]
    t = jnp.maximum(t, 0.0)
    u = jnp.dot(t, w2_ref[...], **_DOT)
    u = a2_ref[1:2, :] * (u + a2_ref[0:1, :]) + a2_ref[2:3, :]
    vn_new = jnp.maximum(u, 0.0)
    vnout_ref[...] = vn_new
    mask = (b2d_ref[...] ==
            lax.broadcasted_iota(jnp.int32, (RB, G), 1)).astype(jnp.float32)
    hnext_ref[...] = hn_ref[...] + jnp.dot(mask, vn_new, **_DOT)


def _run_vn(pool, vn, w1, a1, w2, a2, h_new, batch2d):
    in_specs = [
        pl.BlockSpec((G, D), lambda i: (0, 0)),
        pl.BlockSpec((G, D), lambda i: (0, 0)),
        pl.BlockSpec((D, 2 * D), lambda i: (0, 0)),
        pl.BlockSpec((8, 2 * D), lambda i: (0, 0)),
        pl.BlockSpec((2 * D, D), lambda i: (0, 0)),
        pl.BlockSpec((8, D), lambda i: (0, 0)),
        pl.BlockSpec((RB, D), lambda i: (i, 0)),
        pl.BlockSpec((RB, 1), lambda i: (i, 0)),
    ]
    out_specs = [
        pl.BlockSpec((RB, D), lambda i: (i, 0)),
        pl.BlockSpec((G, D), lambda i: (0, 0)),
    ]
    out_shape = [
        jax.ShapeDtypeStruct((N, D), jnp.float32),
        jax.ShapeDtypeStruct((G, D), jnp.float32),
    ]
    return pl.pallas_call(
        _vn_body, grid=(NBLK,), in_specs=in_specs, out_specs=out_specs,
        out_shape=out_shape,
    )(pool, vn, w1, a1, w2, a2, h_new, batch2d)


def _pred_body(mx_ref, mor_ref, mac_ref, w1h_ref, w1m_ref, w1c_ref,
               b1_ref, w2_ref, b2_ref, out_ref):
    hrep = jnp.max(mx_ref[...], axis=0)
    z = (jnp.dot(hrep, w1h_ref[...], **_DOT)
         + jnp.dot(mor_ref[...], w1m_ref[...], **_DOT)
         + jnp.dot(mac_ref[...], w1c_ref[...], **_DOT)
         + b1_ref[0:1, :])
    z = jnp.maximum(z, 0.0)
    out_ref[...] = jnp.dot(z, w2_ref[...], **_DOT) + b2_ref[0:1, :]


def _pad_rows(v, rows=8):
    v2 = v.reshape(1, -1)
    return jnp.concatenate(
        [v2, jnp.zeros((rows - 1, v2.shape[1]), jnp.float32)], axis=0)


def _aff(b, g, bb, extra=None):
    rows = [b, g, bb] + ([] if extra is None else [extra])
    m = jnp.stack(rows)
    pad = 8 - m.shape[0]
    return jnp.concatenate([m, jnp.zeros((pad, m.shape[1]), jnp.float32)], 0)


def kernel(x, edge_index, batch, morgan, maccs,
           gin_W1, gin_b1, gin_bn1_g, gin_bn1_b, gin_W2, gin_b2, gin_eps,
           bn_g, bn_b, vn_emb, vn_W1, vn_b1, vn_bn1_g, vn_bn1_b,
           vn_W2, vn_b2, vn_bn2_g, vn_bn2_b,
           pred_W1, pred_b1, pred_W2, pred_b2):
    src = edge_index[0]
    dst = edge_index[1]
    batch2d = batch.reshape(N, 1)

    vn = jnp.broadcast_to(vn_emb, (G, D))
    h_prev = x
    h = x + vn_emb[None, :]

    L = gin_W1.shape[0]
    for l in range(L):
        last = l == L - 1
        agg = _edge_agg(h, src, dst)
        a1 = _aff(gin_b1[l], gin_bn1_g[l], gin_bn1_b[l])
        a2 = _aff(gin_b2[l], bn_g[l], bn_b[l],
                  jnp.broadcast_to(1.0 + gin_eps[l], (D,)))
        if last:
            (h_new,) = _run_gin(h, h_prev, agg, batch2d,
                                gin_W1[l], a1, gin_W2[l], a2, last=True)
            h_prev = h_new
        else:
            h_new, pool = _run_gin(h, h_prev, agg, batch2d,
                                   gin_W1[l], a1, gin_W2[l], a2, last=False)
            av1 = _aff(vn_b1[l], vn_bn1_g[l], vn_bn1_b[l])
            av2 = _aff(vn_b2[l], vn_bn2_g[l], vn_bn2_b[l])
            h, vn = _run_vn(pool, vn, vn_W1[l], av1, vn_W2[l], av2,
                            h_new, batch2d)
            h_prev = h_new

    mx = _segmax(h_prev, batch)

    w1h = pred_W1[0:D]
    w1m = pred_W1[D:D + 1024]
    w1c = jnp.concatenate(
        [pred_W1[D + 1024:], jnp.zeros((256 - 167, 2 * D), jnp.float32)], 0)
    mac_p = jnp.concatenate(
        [maccs, jnp.zeros((G, 256 - 167), jnp.float32)], 1)
    b1p = _pad_rows(pred_b1)
    w2p = jnp.concatenate(
        [pred_W2, jnp.zeros((2 * D, 128 - NUM_TASK), jnp.float32)], 1)
    b2p = _pad_rows(jnp.concatenate(
        [pred_b2, jnp.zeros((128 - NUM_TASK,), jnp.float32)]))

    out = pl.pallas_call(
        _pred_body,
        out_shape=jax.ShapeDtypeStruct((G, 128), jnp.float32),
    )(mx, morgan, mac_p, w1h, w1m, w1c, b1p, w2p, b2p)
    return out[:, :NUM_TASK]


# speedup vs baseline: 7.2757x; 1.6057x over previous
import functools

import jax
import jax.numpy as jnp
from jax import lax
from jax.experimental import pallas as pl
from jax.experimental.pallas import tpu as pltpu
from jax.experimental.pallas import tpu_sc as plsc

N = 10000
E = 320000
D = 128
G = 128
NUM_TASK = 10

NC = 2
NS = 16
NW = NC * NS
CHUNK = 64
NCHUNKS = E // CHUNK
TILE_CHUNKS = NCHUNKS // NW
NBUF = 4
NGRP = TILE_CHUNKS // NBUF
NLEFT = NCHUNKS - NW * TILE_CHUNKS
ROWS_PER_TILE = 632
SEG = 312
NEG = -3.4028235e38


def _edge_agg_body(h_hbm, src_hbm, dst_hbm, out_hbm,
                   src_v, dst_v, rows_v, acc_sh, *sems):
    gsems = sems[:NBUF]
    isem_s, isem_d = sems[NBUF], sems[NBUF + 1]
    c = lax.axis_index("c")
    s = lax.axis_index("s")
    wid = c * NS + s
    crow = wid * TILE_CHUNKS

    pltpu.sync_copy(src_hbm.at[pl.ds(crow, NBUF)], src_v.at[0])
    pltpu.sync_copy(dst_hbm.at[pl.ds(crow, NBUF)], dst_v.at[0])

    def _zrow(i, carry):
        for r in range(8):
            rows_v[0, i, pl.ds(r * 16, 16)] = jnp.zeros((16,), jnp.float32)
        return carry
    lax.fori_loop(0, CHUNK, _zrow, 0)
    base0 = s * ROWS_PER_TILE
    nfull = jnp.where(s < NS - 1, 9, 8)

    def _zcopy(k, carry):
        pltpu.sync_copy(rows_v.at[0],
                        acc_sh.at[pl.ds(base0 + k * CHUNK, CHUNK)])
        return carry
    lax.fori_loop(0, nfull, _zcopy, 0)

    @pl.when(s < NS - 1)
    def _():
        pltpu.sync_copy(rows_v.at[0, pl.ds(0, 56)],
                        acc_sh.at[pl.ds(base0 + 576, 56)])

    @pl.when(s == NS - 1)
    def _():
        pltpu.sync_copy(rows_v.at[0, pl.ds(0, 8)],
                        acc_sh.at[pl.ds(base0 + 512, 8)])

    plsc.subcore_barrier()

    def _group(g, carry):
        pb = lax.rem(g, 2)
        nb = 1 - pb
        nrow = crow + (g + 1) * NBUF

        @pl.when(g < NGRP - 1)
        def _():
            pltpu.async_copy(src_hbm.at[pl.ds(nrow, NBUF)], src_v.at[nb],
                             isem_s)
            pltpu.async_copy(dst_hbm.at[pl.ds(nrow, NBUF)], dst_v.at[nb],
                             isem_d)

        descs = []
        for b in range(NBUF):
            descs.append(pltpu.async_copy(
                h_hbm.at[src_v.at[pb, b, 0]], rows_v.at[b], gsems[b]))
        for b in range(NBUF):
            descs[b].wait()
            pltpu.sync_copy(rows_v.at[b], acc_sh.at[dst_v.at[pb, b, 0]],
                            add=True)

        @pl.when(g < NGRP - 1)
        def _():
            pltpu.make_async_copy(src_hbm.at[pl.ds(nrow, NBUF)],
                                  src_v.at[nb], isem_s).wait()
            pltpu.make_async_copy(dst_hbm.at[pl.ds(nrow, NBUF)],
                                  dst_v.at[nb], isem_d).wait()
        return carry
    lax.fori_loop(0, NGRP, _group, 0)

    @pl.when(wid < NLEFT)
    def _():
        lrow = NW * TILE_CHUNKS + wid
        pltpu.sync_copy(src_hbm.at[pl.ds(lrow, 1)], src_v.at[0, pl.ds(0, 1)])
        pltpu.sync_copy(dst_hbm.at[pl.ds(lrow, 1)], dst_v.at[0, pl.ds(0, 1)])
        pltpu.async_copy(h_hbm.at[src_v.at[0, 0, 0]], rows_v.at[0],
                         gsems[0]).wait()
        pltpu.sync_copy(rows_v.at[0], acc_sh.at[dst_v.at[0, 0, 0]], add=True)

    plsc.subcore_barrier()

    def _drain(k, carry):
        b = base0 + k * CHUNK
        pltpu.sync_copy(acc_sh.at[pl.ds(b, CHUNK)], rows_v.at[0])
        pltpu.sync_copy(rows_v.at[0], out_hbm.at[c, pl.ds(b, CHUNK)])
        return carry
    lax.fori_loop(0, nfull, _drain, 0)

    @pl.when(s < NS - 1)
    def _():
        pltpu.sync_copy(acc_sh.at[pl.ds(base0 + 576, 56)],
                        rows_v.at[0, pl.ds(0, 56)])
        pltpu.sync_copy(rows_v.at[0, pl.ds(0, 56)],
                        out_hbm.at[c, pl.ds(base0 + 576, 56)])

    @pl.when(s == NS - 1)
    def _():
        pltpu.sync_copy(acc_sh.at[pl.ds(base0 + 512, 8)],
                        rows_v.at[0, pl.ds(0, 8)])
        pltpu.sync_copy(rows_v.at[0, pl.ds(0, 8)],
                        out_hbm.at[c, pl.ds(base0 + 512, 8)])


@functools.cache
def _edge_agg_kernel():
    mesh = plsc.VectorSubcoreMesh(core_axis_name="c", subcore_axis_name="s")
    return pl.kernel(
        _edge_agg_body,
        out_type=jax.ShapeDtypeStruct((NC, N, D), jnp.float32),
        mesh=mesh,
        scratch_types=[
            pltpu.VMEM((2, NBUF, 1, CHUNK), jnp.int32),
            pltpu.VMEM((2, NBUF, 1, CHUNK), jnp.int32),
            pltpu.VMEM((NBUF, CHUNK, D), jnp.float32),
            pltpu.VMEM_SHARED((N, D), jnp.float32),
        ] + [pltpu.SemaphoreType.DMA] * (NBUF + 2),
    )


def _edge_agg(h, src, dst):
    src2 = src.reshape(NCHUNKS, 1, CHUNK)
    dst2 = dst.reshape(NCHUNKS, 1, CHUNK)
    return _edge_agg_kernel()(h, src2, dst2)


def _segmax_body(h_hbm, batch_hbm, out_hbm, batch_v, rows_v, acc_v):
    c = lax.axis_index("c")
    s = lax.axis_index("s")
    wid = c * NS + s
    base = wid * SEG

    pltpu.sync_copy(batch_hbm.at[pl.ds(base, SEG)], batch_v.at[pl.ds(0, SEG)])
    pltpu.sync_copy(h_hbm.at[pl.ds(base, SEG)], rows_v.at[pl.ds(0, SEG)])

    @pl.when(wid == NW - 1)
    def _():
        pltpu.sync_copy(batch_hbm.at[pl.ds(N - 16, 16)],
                        batch_v.at[pl.ds(SEG, 16)])
        pltpu.sync_copy(h_hbm.at[pl.ds(N - 16, 16)],
                        rows_v.at[pl.ds(SEG, 16)])

    neg = jnp.full((16,), NEG, jnp.float32)

    def _irow(i, carry):
        for r in range(8):
            acc_v[i, pl.ds(r * 16, 16)] = neg
        return carry
    lax.fori_loop(0, G, _irow, 0)

    n = SEG + jnp.where(wid == NW - 1, 16, 0)

    def _node(i, carry):
        g = batch_v[pl.ds(i, 16)][0]
        for r in range(8):
            sl = pl.ds(r * 16, 16)
            acc_v[g, sl] = jnp.maximum(acc_v[g, sl], rows_v[i, sl])
        return carry
    lax.fori_loop(0, n, _node, 0)

    pltpu.sync_copy(acc_v, out_hbm.at[wid])


@functools.cache
def _segmax_kernel():
    mesh = plsc.VectorSubcoreMesh(core_axis_name="c", subcore_axis_name="s")
    return pl.kernel(
        _segmax_body,
        out_type=jax.ShapeDtypeStruct((NW, G, D), jnp.float32),
        mesh=mesh,
        scratch_types=[
            pltpu.VMEM((SEG + 16 + 16,), jnp.int32),
            pltpu.VMEM((SEG + 16, D), jnp.float32),
            pltpu.VMEM((G, D), jnp.float32),
        ],
    )


def _segmax(h, batch):
    return _segmax_kernel()(h, batch)


RB = 1000
NBLK = N // RB

def _mask_dot(mask, v, pool=False):
    m = mask.astype(jnp.bfloat16)
    v1 = v.astype(jnp.bfloat16)
    v1f = v1.astype(jnp.float32)
    v2 = (v - v1f).astype(jnp.bfloat16)
    v2f = v2.astype(jnp.float32)
    v3 = (v - v1f - v2f).astype(jnp.bfloat16)
    if pool:
        dn = (((0,), (0,)), ((), ()))
        parts = [lax.dot_general(m, t, dn, preferred_element_type=jnp.float32)
                 for t in (v1, v2, v3)]
    else:
        parts = [jnp.dot(m, t, preferred_element_type=jnp.float32)
                 for t in (v1, v2, v3)]
    return parts[0] + parts[1] + parts[2]


def _bdot(a, b):
    return jnp.dot(a.astype(jnp.bfloat16), b.astype(jnp.bfloat16),
                   preferred_element_type=jnp.float32)


def _xdot(a, b):
    return jnp.dot(a, b, preferred_element_type=jnp.float32,
                   precision=lax.Precision.HIGHEST)


def _gin_body(*refs, last):
    if last:
        h_ref, hp_ref, agg_ref, b2d_ref, w1_ref, a1_ref, w2_ref, a2_ref, out_ref = refs
    else:
        (h_ref, hp_ref, agg_ref, b2d_ref, w1_ref, a1_ref, w2_ref, a2_ref,
         out_ref, pool_ref) = refs
    h = h_ref[...]
    hp = hp_ref[...]
    z0 = a2_ref[3:4, :] * h + agg_ref[0] + agg_ref[1]
    t = _bdot(z0, w1_ref[...])
    t = a1_ref[1:2, :] * (t + a1_ref[0:1, :]) + a1_ref[2:3, :]
    t = jnp.maximum(t, 0.0)
    u = _bdot(t, w2_ref[...])
    u = a2_ref[1:2, :] * (u + a2_ref[0:1, :]) + a2_ref[2:3, :]
    if not last:
        u = jnp.maximum(u, 0.0)
    out_ref[...] = u + hp
    if not last:
        i = pl.program_id(0)
        mask = (b2d_ref[...] ==
                lax.broadcasted_iota(jnp.int32, (RB, G), 1)).astype(jnp.float32)
        p = _mask_dot(mask, hp, pool=True)

        @pl.when(i == 0)
        def _():
            pool_ref[...] = p

        @pl.when(i > 0)
        def _():
            pool_ref[...] += p


def _run_gin(h, h_prev, agg, batch2d, w1, a1, w2, a2, last):
    body = functools.partial(_gin_body, last=last)
    out_shape = [jax.ShapeDtypeStruct((N, D), jnp.float32)]
    out_specs = [pl.BlockSpec((RB, D), lambda i: (i, 0))]
    if not last:
        out_shape.append(jax.ShapeDtypeStruct((G, D), jnp.float32))
        out_specs.append(pl.BlockSpec((G, D), lambda i: (0, 0)))
    in_specs = [
        pl.BlockSpec((RB, D), lambda i: (i, 0)),
        pl.BlockSpec((RB, D), lambda i: (i, 0)),
        pl.BlockSpec((NC, RB, D), lambda i: (0, i, 0)),
        pl.BlockSpec((RB, 1), lambda i: (i, 0)),
        pl.BlockSpec((D, 2 * D), lambda i: (0, 0)),
        pl.BlockSpec((8, 2 * D), lambda i: (0, 0)),
        pl.BlockSpec((2 * D, D), lambda i: (0, 0)),
        pl.BlockSpec((8, D), lambda i: (0, 0)),
    ]
    return pl.pallas_call(
        body, grid=(NBLK,), in_specs=in_specs, out_specs=out_specs,
        out_shape=out_shape,
    )(h, h_prev, agg, batch2d, w1, a1, w2, a2)


def _vn_body(pool_ref, vn_ref, w1_ref, a1_ref, w2_ref, a2_ref,
             hn_ref, b2d_ref, hnext_ref, vnout_ref):
    vt = pool_ref[...] + vn_ref[...]
    t = _bdot(vt, w1_ref[...])
    t = a1_ref[1:2, :] * (t + a1_ref[0:1, :]) + a1_ref[2:3, :]
    t = jnp.maximum(t, 0.0)
    u = _bdot(t, w2_ref[...])
    u = a2_ref[1:2, :] * (u + a2_ref[0:1, :]) + a2_ref[2:3, :]
    vn_new = jnp.maximum(u, 0.0)
    vnout_ref[...] = vn_new
    mask = (b2d_ref[...] ==
            lax.broadcasted_iota(jnp.int32, (RB, G), 1)).astype(jnp.float32)
    hnext_ref[...] = hn_ref[...] + _mask_dot(mask, vn_new)


def _run_vn(pool, vn, w1, a1, w2, a2, h_new, batch2d):
    in_specs = [
        pl.BlockSpec((G, D), lambda i: (0, 0)),
        pl.BlockSpec((G, D), lambda i: (0, 0)),
        pl.BlockSpec((D, 2 * D), lambda i: (0, 0)),
        pl.BlockSpec((8, 2 * D), lambda i: (0, 0)),
        pl.BlockSpec((2 * D, D), lambda i: (0, 0)),
        pl.BlockSpec((8, D), lambda i: (0, 0)),
        pl.BlockSpec((RB, D), lambda i: (i, 0)),
        pl.BlockSpec((RB, 1), lambda i: (i, 0)),
    ]
    out_specs = [
        pl.BlockSpec((RB, D), lambda i: (i, 0)),
        pl.BlockSpec((G, D), lambda i: (0, 0)),
    ]
    out_shape = [
        jax.ShapeDtypeStruct((N, D), jnp.float32),
        jax.ShapeDtypeStruct((G, D), jnp.float32),
    ]
    return pl.pallas_call(
        _vn_body, grid=(NBLK,), in_specs=in_specs, out_specs=out_specs,
        out_shape=out_shape,
    )(pool, vn, w1, a1, w2, a2, h_new, batch2d)


def _pred_body(mx_ref, mor_ref, mac_ref, w1h_ref, w1m_ref, w1c_ref,
               b1_ref, w2_ref, b2_ref, out_ref):
    hrep = jnp.max(mx_ref[...], axis=0)
    z = (_bdot(hrep, w1h_ref[...])
         + _bdot(mor_ref[...], w1m_ref[...])
         + _bdot(mac_ref[...], w1c_ref[...])
         + b1_ref[0:1, :])
    z = jnp.maximum(z, 0.0)
    out_ref[...] = _bdot(z, w2_ref[...]) + b2_ref[0:1, :]


def _pad_rows(v, rows=8):
    v2 = v.reshape(1, -1)
    return jnp.concatenate(
        [v2, jnp.zeros((rows - 1, v2.shape[1]), jnp.float32)], axis=0)


def _aff(b, g, bb, extra=None):
    rows = [b, g, bb] + ([] if extra is None else [extra])
    m = jnp.stack(rows)
    pad = 8 - m.shape[0]
    return jnp.concatenate([m, jnp.zeros((pad, m.shape[1]), jnp.float32)], 0)


def kernel(x, edge_index, batch, morgan, maccs,
           gin_W1, gin_b1, gin_bn1_g, gin_bn1_b, gin_W2, gin_b2, gin_eps,
           bn_g, bn_b, vn_emb, vn_W1, vn_b1, vn_bn1_g, vn_bn1_b,
           vn_W2, vn_b2, vn_bn2_g, vn_bn2_b,
           pred_W1, pred_b1, pred_W2, pred_b2):
    src = edge_index[0]
    dst = edge_index[1]
    batch2d = batch.reshape(N, 1)

    vn = jnp.broadcast_to(vn_emb, (G, D))
    h_prev = x
    h = x + vn_emb[None, :]

    L = gin_W1.shape[0]
    for l in range(L):
        last = l == L - 1
        agg = _edge_agg(h, src, dst)
        a1 = _aff(gin_b1[l], gin_bn1_g[l], gin_bn1_b[l])
        a2 = _aff(gin_b2[l], bn_g[l], bn_b[l],
                  jnp.broadcast_to(1.0 + gin_eps[l], (D,)))
        if last:
            (h_new,) = _run_gin(h, h_prev, agg, batch2d,
                                gin_W1[l], a1, gin_W2[l], a2, last=True)
            h_prev = h_new
        else:
            h_new, pool = _run_gin(h, h_prev, agg, batch2d,
                                   gin_W1[l], a1, gin_W2[l], a2, last=False)
            av1 = _aff(vn_b1[l], vn_bn1_g[l], vn_bn1_b[l])
            av2 = _aff(vn_b2[l], vn_bn2_g[l], vn_bn2_b[l])
            h, vn = _run_vn(pool, vn, vn_W1[l], av1, vn_W2[l], av2,
                            h_new, batch2d)
            h_prev = h_new

    mx = _segmax(h_prev, batch)

    w1h = pred_W1[0:D]
    w1m = pred_W1[D:D + 1024]
    w1c = jnp.concatenate(
        [pred_W1[D + 1024:], jnp.zeros((256 - 167, 2 * D), jnp.float32)], 0)
    mac_p = jnp.concatenate(
        [maccs, jnp.zeros((G, 256 - 167), jnp.float32)], 1)
    b1p = _pad_rows(pred_b1)
    w2p = jnp.concatenate(
        [pred_W2, jnp.zeros((2 * D, 128 - NUM_TASK), jnp.float32)], 1)
    b2p = _pad_rows(jnp.concatenate(
        [pred_b2, jnp.zeros((128 - NUM_TASK,), jnp.float32)]))

    out = pl.pallas_call(
        _pred_body,
        out_shape=jax.ShapeDtypeStruct((G, 128), jnp.float32),
    )(mx, morgan, mac_p, w1h, w1m, w1c, b1p, w2p, b2p)
    return out[:, :NUM_TASK]


# speedup vs baseline: 9.1665x; 1.2599x over previous
import functools

import jax
import jax.numpy as jnp
from jax import lax
from jax.experimental import pallas as pl
from jax.experimental.pallas import tpu as pltpu
from jax.experimental.pallas import tpu_sc as plsc

N = 10000
E = 320000
D = 128
G = 128
NUM_TASK = 10

NC = 2
NS = 16
NW = NC * NS
CHUNK = 64
NCHUNKS = E // CHUNK
TILE_CHUNKS = NCHUNKS // NW
NBUF = 4
NGRP = TILE_CHUNKS // NBUF
NLEFT = NCHUNKS - NW * TILE_CHUNKS
ROWS_PER_TILE = 632
SEG = 312
NEG = -3.4028235e38


def _edge_agg_body(h_hbm, src_hbm, dst_hbm, out_hbm,
                   src_v, dst_v, rows_v, acc_sh, *sems):
    gsems = sems[:NBUF]
    ssems = sems[NBUF:2 * NBUF]
    isem_s, isem_d = sems[2 * NBUF], sems[2 * NBUF + 1]
    c = lax.axis_index("c")
    s = lax.axis_index("s")
    wid = c * NS + s
    crow = wid * TILE_CHUNKS

    pltpu.sync_copy(src_hbm.at[pl.ds(crow, NBUF)], src_v.at[0])
    pltpu.sync_copy(dst_hbm.at[pl.ds(crow, NBUF)], dst_v.at[0])

    def _zrow(i, carry):
        for r in range(8):
            rows_v[0, i, pl.ds(r * 16, 16)] = jnp.zeros((16,), jnp.float32)
        return carry
    lax.fori_loop(0, CHUNK, _zrow, 0)
    base0 = s * ROWS_PER_TILE
    nfull = jnp.where(s < NS - 1, 9, 8)

    def _zcopy(k, carry):
        pltpu.sync_copy(rows_v.at[0],
                        acc_sh.at[pl.ds(base0 + k * CHUNK, CHUNK)])
        return carry
    lax.fori_loop(0, nfull, _zcopy, 0)

    @pl.when(s < NS - 1)
    def _():
        pltpu.sync_copy(rows_v.at[0, pl.ds(0, 56)],
                        acc_sh.at[pl.ds(base0 + 576, 56)])

    @pl.when(s == NS - 1)
    def _():
        pltpu.sync_copy(rows_v.at[0, pl.ds(0, 8)],
                        acc_sh.at[pl.ds(base0 + 512, 8)])

    plsc.subcore_barrier()

    def _group(g, carry):
        pb = lax.rem(g, 2)
        nb = 1 - pb
        nrow = crow + (g + 1) * NBUF

        @pl.when(g < NGRP - 1)
        def _():
            pltpu.async_copy(src_hbm.at[pl.ds(nrow, NBUF)], src_v.at[nb],
                             isem_s)
            pltpu.async_copy(dst_hbm.at[pl.ds(nrow, NBUF)], dst_v.at[nb],
                             isem_d)

        descs = []
        for b in range(NBUF):
            @pl.when(g > 0)
            def _(b=b):
                pltpu.make_async_copy(rows_v.at[b],
                                      acc_sh.at[dst_v.at[nb, b, 0]],
                                      ssems[b]).wait()
            descs.append(pltpu.async_copy(
                h_hbm.at[src_v.at[pb, b, 0]], rows_v.at[b], gsems[b]))
        for b in range(NBUF):
            descs[b].wait()
            pltpu.async_copy(rows_v.at[b], acc_sh.at[dst_v.at[pb, b, 0]],
                             ssems[b], add=True)

        @pl.when(g < NGRP - 1)
        def _():
            pltpu.make_async_copy(src_hbm.at[pl.ds(nrow, NBUF)],
                                  src_v.at[nb], isem_s).wait()
            pltpu.make_async_copy(dst_hbm.at[pl.ds(nrow, NBUF)],
                                  dst_v.at[nb], isem_d).wait()
        return carry
    lax.fori_loop(0, NGRP, _group, 0)

    last_pb = (NGRP - 1) % 2
    for b in range(NBUF):
        pltpu.make_async_copy(rows_v.at[b],
                              acc_sh.at[dst_v.at[last_pb, b, 0]],
                              ssems[b]).wait()

    @pl.when(wid < NLEFT)
    def _():
        lrow = NW * TILE_CHUNKS + wid
        pltpu.sync_copy(src_hbm.at[pl.ds(lrow, 1)], src_v.at[0, pl.ds(0, 1)])
        pltpu.sync_copy(dst_hbm.at[pl.ds(lrow, 1)], dst_v.at[0, pl.ds(0, 1)])
        pltpu.async_copy(h_hbm.at[src_v.at[0, 0, 0]], rows_v.at[0],
                         gsems[0]).wait()
        pltpu.sync_copy(rows_v.at[0], acc_sh.at[dst_v.at[0, 0, 0]], add=True)

    plsc.subcore_barrier()

    def _drain(k, carry):
        b = base0 + k * CHUNK
        pltpu.sync_copy(acc_sh.at[pl.ds(b, CHUNK)], rows_v.at[0])
        pltpu.sync_copy(rows_v.at[0], out_hbm.at[c, pl.ds(b, CHUNK)])
        return carry
    lax.fori_loop(0, nfull, _drain, 0)

    @pl.when(s < NS - 1)
    def _():
        pltpu.sync_copy(acc_sh.at[pl.ds(base0 + 576, 56)],
                        rows_v.at[0, pl.ds(0, 56)])
        pltpu.sync_copy(rows_v.at[0, pl.ds(0, 56)],
                        out_hbm.at[c, pl.ds(base0 + 576, 56)])

    @pl.when(s == NS - 1)
    def _():
        pltpu.sync_copy(acc_sh.at[pl.ds(base0 + 512, 8)],
                        rows_v.at[0, pl.ds(0, 8)])
        pltpu.sync_copy(rows_v.at[0, pl.ds(0, 8)],
                        out_hbm.at[c, pl.ds(base0 + 512, 8)])


@functools.cache
def _edge_agg_kernel():
    mesh = plsc.VectorSubcoreMesh(core_axis_name="c", subcore_axis_name="s")
    return pl.kernel(
        _edge_agg_body,
        out_type=jax.ShapeDtypeStruct((NC, N, D), jnp.float32),
        mesh=mesh,
        scratch_types=[
            pltpu.VMEM((2, NBUF, 1, CHUNK), jnp.int32),
            pltpu.VMEM((2, NBUF, 1, CHUNK), jnp.int32),
            pltpu.VMEM((NBUF, CHUNK, D), jnp.float32),
            pltpu.VMEM_SHARED((N, D), jnp.float32),
        ] + [pltpu.SemaphoreType.DMA] * (2 * NBUF + 2),
    )


def _edge_agg(h, src, dst):
    src2 = src.reshape(NCHUNKS, 1, CHUNK)
    dst2 = dst.reshape(NCHUNKS, 1, CHUNK)
    return _edge_agg_kernel()(h, src2, dst2)


def _segmax_body(h_hbm, batch_hbm, out_hbm, batch_v, rows_v, acc_v):
    c = lax.axis_index("c")
    s = lax.axis_index("s")
    wid = c * NS + s
    base = wid * SEG

    pltpu.sync_copy(batch_hbm.at[pl.ds(base, SEG)], batch_v.at[pl.ds(0, SEG)])
    pltpu.sync_copy(h_hbm.at[pl.ds(base, SEG)], rows_v.at[pl.ds(0, SEG)])

    @pl.when(wid == NW - 1)
    def _():
        pltpu.sync_copy(batch_hbm.at[pl.ds(N - 16, 16)],
                        batch_v.at[pl.ds(SEG, 16)])
        pltpu.sync_copy(h_hbm.at[pl.ds(N - 16, 16)],
                        rows_v.at[pl.ds(SEG, 16)])

    neg = jnp.full((16,), NEG, jnp.float32)

    def _irow(i, carry):
        for r in range(8):
            acc_v[i, pl.ds(r * 16, 16)] = neg
        return carry
    lax.fori_loop(0, G, _irow, 0)

    n = SEG + jnp.where(wid == NW - 1, 16, 0)

    def _node(i, carry):
        g = batch_v[pl.ds(i, 16)][0]
        for r in range(8):
            sl = pl.ds(r * 16, 16)
            acc_v[g, sl] = jnp.maximum(acc_v[g, sl], rows_v[i, sl])
        return carry
    lax.fori_loop(0, n, _node, 0)

    pltpu.sync_copy(acc_v, out_hbm.at[wid])


@functools.cache
def _segmax_kernel():
    mesh = plsc.VectorSubcoreMesh(core_axis_name="c", subcore_axis_name="s")
    return pl.kernel(
        _segmax_body,
        out_type=jax.ShapeDtypeStruct((NW, G, D), jnp.float32),
        mesh=mesh,
        scratch_types=[
            pltpu.VMEM((SEG + 16 + 16,), jnp.int32),
            pltpu.VMEM((SEG + 16, D), jnp.float32),
            pltpu.VMEM((G, D), jnp.float32),
        ],
    )


def _segmax(h, batch):
    return _segmax_kernel()(h, batch)


RB = 1000
NBLK = N // RB

def _mask_dot(mask, v, pool=False):
    m = mask.astype(jnp.bfloat16)
    v1 = v.astype(jnp.bfloat16)
    v1f = v1.astype(jnp.float32)
    v2 = (v - v1f).astype(jnp.bfloat16)
    v2f = v2.astype(jnp.float32)
    v3 = (v - v1f - v2f).astype(jnp.bfloat16)
    if pool:
        dn = (((0,), (0,)), ((), ()))
        parts = [lax.dot_general(m, t, dn, preferred_element_type=jnp.float32)
                 for t in (v1, v2, v3)]
    else:
        parts = [jnp.dot(m, t, preferred_element_type=jnp.float32)
                 for t in (v1, v2, v3)]
    return parts[0] + parts[1] + parts[2]


def _bdot(a, b):
    return jnp.dot(a.astype(jnp.bfloat16), b.astype(jnp.bfloat16),
                   preferred_element_type=jnp.float32)


def _xdot(a, b):
    return jnp.dot(a, b, preferred_element_type=jnp.float32,
                   precision=lax.Precision.HIGHEST)


def _gin_body(*refs, last):
    if last:
        h_ref, hp_ref, agg_ref, b2d_ref, w1_ref, a1_ref, w2_ref, a2_ref, out_ref = refs
    else:
        (h_ref, hp_ref, agg_ref, b2d_ref, w1_ref, a1_ref, w2_ref, a2_ref,
         out_ref, pool_ref) = refs
    h = h_ref[...]
    hp = hp_ref[...]
    z0 = a2_ref[3:4, :] * h + agg_ref[0] + agg_ref[1]
    t = _bdot(z0, w1_ref[...])
    t = a1_ref[1:2, :] * (t + a1_ref[0:1, :]) + a1_ref[2:3, :]
    t = jnp.maximum(t, 0.0)
    u = _bdot(t, w2_ref[...])
    u = a2_ref[1:2, :] * (u + a2_ref[0:1, :]) + a2_ref[2:3, :]
    if not last:
        u = jnp.maximum(u, 0.0)
    out_ref[...] = u + hp
    if not last:
        i = pl.program_id(0)
        mask = (b2d_ref[...] ==
                lax.broadcasted_iota(jnp.int32, (RB, G), 1)).astype(jnp.float32)
        p = _mask_dot(mask, hp, pool=True)

        @pl.when(i == 0)
        def _():
            pool_ref[...] = p

        @pl.when(i > 0)
        def _():
            pool_ref[...] += p


def _run_gin(h, h_prev, agg, batch2d, w1, a1, w2, a2, last):
    body = functools.partial(_gin_body, last=last)
    out_shape = [jax.ShapeDtypeStruct((N, D), jnp.float32)]
    out_specs = [pl.BlockSpec((RB, D), lambda i: (i, 0))]
    if not last:
        out_shape.append(jax.ShapeDtypeStruct((G, D), jnp.float32))
        out_specs.append(pl.BlockSpec((G, D), lambda i: (0, 0)))
    in_specs = [
        pl.BlockSpec((RB, D), lambda i: (i, 0)),
        pl.BlockSpec((RB, D), lambda i: (i, 0)),
        pl.BlockSpec((NC, RB, D), lambda i: (0, i, 0)),
        pl.BlockSpec((RB, 1), lambda i: (i, 0)),
        pl.BlockSpec((D, 2 * D), lambda i: (0, 0)),
        pl.BlockSpec((8, 2 * D), lambda i: (0, 0)),
        pl.BlockSpec((2 * D, D), lambda i: (0, 0)),
        pl.BlockSpec((8, D), lambda i: (0, 0)),
    ]
    return pl.pallas_call(
        body, grid=(NBLK,), in_specs=in_specs, out_specs=out_specs,
        out_shape=out_shape,
    )(h, h_prev, agg, batch2d, w1, a1, w2, a2)


def _vn_body(pool_ref, vn_ref, w1_ref, a1_ref, w2_ref, a2_ref,
             hn_ref, b2d_ref, hnext_ref, vnout_ref):
    vt = pool_ref[...] + vn_ref[...]
    t = _bdot(vt, w1_ref[...])
    t = a1_ref[1:2, :] * (t + a1_ref[0:1, :]) + a1_ref[2:3, :]
    t = jnp.maximum(t, 0.0)
    u = _bdot(t, w2_ref[...])
    u = a2_ref[1:2, :] * (u + a2_ref[0:1, :]) + a2_ref[2:3, :]
    vn_new = jnp.maximum(u, 0.0)
    vnout_ref[...] = vn_new
    mask = (b2d_ref[...] ==
            lax.broadcasted_iota(jnp.int32, (RB, G), 1)).astype(jnp.float32)
    hnext_ref[...] = hn_ref[...] + _mask_dot(mask, vn_new)


def _run_vn(pool, vn, w1, a1, w2, a2, h_new, batch2d):
    in_specs = [
        pl.BlockSpec((G, D), lambda i: (0, 0)),
        pl.BlockSpec((G, D), lambda i: (0, 0)),
        pl.BlockSpec((D, 2 * D), lambda i: (0, 0)),
        pl.BlockSpec((8, 2 * D), lambda i: (0, 0)),
        pl.BlockSpec((2 * D, D), lambda i: (0, 0)),
        pl.BlockSpec((8, D), lambda i: (0, 0)),
        pl.BlockSpec((RB, D), lambda i: (i, 0)),
        pl.BlockSpec((RB, 1), lambda i: (i, 0)),
    ]
    out_specs = [
        pl.BlockSpec((RB, D), lambda i: (i, 0)),
        pl.BlockSpec((G, D), lambda i: (0, 0)),
    ]
    out_shape = [
        jax.ShapeDtypeStruct((N, D), jnp.float32),
        jax.ShapeDtypeStruct((G, D), jnp.float32),
    ]
    return pl.pallas_call(
        _vn_body, grid=(NBLK,), in_specs=in_specs, out_specs=out_specs,
        out_shape=out_shape,
    )(pool, vn, w1, a1, w2, a2, h_new, batch2d)


def _pred_body(mx_ref, mor_ref, mac_ref, w1h_ref, w1m_ref, w1c_ref,
               b1_ref, w2_ref, b2_ref, out_ref):
    hrep = jnp.max(mx_ref[...], axis=0)
    z = (_bdot(hrep, w1h_ref[...])
         + _bdot(mor_ref[...], w1m_ref[...])
         + _bdot(mac_ref[...], w1c_ref[...])
         + b1_ref[0:1, :])
    z = jnp.maximum(z, 0.0)
    out_ref[...] = _bdot(z, w2_ref[...]) + b2_ref[0:1, :]


def _pad_rows(v, rows=8):
    v2 = v.reshape(1, -1)
    return jnp.concatenate(
        [v2, jnp.zeros((rows - 1, v2.shape[1]), jnp.float32)], axis=0)


def _aff(b, g, bb, extra=None):
    rows = [b, g, bb] + ([] if extra is None else [extra])
    m = jnp.stack(rows)
    pad = 8 - m.shape[0]
    return jnp.concatenate([m, jnp.zeros((pad, m.shape[1]), jnp.float32)], 0)


def kernel(x, edge_index, batch, morgan, maccs,
           gin_W1, gin_b1, gin_bn1_g, gin_bn1_b, gin_W2, gin_b2, gin_eps,
           bn_g, bn_b, vn_emb, vn_W1, vn_b1, vn_bn1_g, vn_bn1_b,
           vn_W2, vn_b2, vn_bn2_g, vn_bn2_b,
           pred_W1, pred_b1, pred_W2, pred_b2):
    src = edge_index[0]
    dst = edge_index[1]
    batch2d = batch.reshape(N, 1)

    vn = jnp.broadcast_to(vn_emb, (G, D))
    h_prev = x
    h = x + vn_emb[None, :]

    L = gin_W1.shape[0]
    for l in range(L):
        last = l == L - 1
        agg = _edge_agg(h, src, dst)
        a1 = _aff(gin_b1[l], gin_bn1_g[l], gin_bn1_b[l])
        a2 = _aff(gin_b2[l], bn_g[l], bn_b[l],
                  jnp.broadcast_to(1.0 + gin_eps[l], (D,)))
        if last:
            (h_new,) = _run_gin(h, h_prev, agg, batch2d,
                                gin_W1[l], a1, gin_W2[l], a2, last=True)
            h_prev = h_new
        else:
            h_new, pool = _run_gin(h, h_prev, agg, batch2d,
                                   gin_W1[l], a1, gin_W2[l], a2, last=False)
            av1 = _aff(vn_b1[l], vn_bn1_g[l], vn_bn1_b[l])
            av2 = _aff(vn_b2[l], vn_bn2_g[l], vn_bn2_b[l])
            h, vn = _run_vn(pool, vn, vn_W1[l], av1, vn_W2[l], av2,
                            h_new, batch2d)
            h_prev = h_new

    mx = _segmax(h_prev, batch)

    w1h = pred_W1[0:D]
    w1m = pred_W1[D:D + 1024]
    w1c = jnp.concatenate(
        [pred_W1[D + 1024:], jnp.zeros((256 - 167, 2 * D), jnp.float32)], 0)
    mac_p = jnp.concatenate(
        [maccs, jnp.zeros((G, 256 - 167), jnp.float32)], 1)
    b1p = _pad_rows(pred_b1)
    w2p = jnp.concatenate(
        [pred_W2, jnp.zeros((2 * D, 128 - NUM_TASK), jnp.float32)], 1)
    b2p = _pad_rows(jnp.concatenate(
        [pred_b2, jnp.zeros((128 - NUM_TASK,), jnp.float32)]))

    out = pl.pallas_call(
        _pred_body,
        out_shape=jax.ShapeDtypeStruct((G, 128), jnp.float32),
    )(mx, morgan, mac_p, w1h, w1m, w1c, b1p, w2p, b2p)
    return out[:, :NUM_TASK]


# speedup vs baseline: 9.2024x; 1.0039x over previous
import functools

import jax
import jax.numpy as jnp
from jax import lax
from jax.experimental import pallas as pl
from jax.experimental.pallas import tpu as pltpu
from jax.experimental.pallas import tpu_sc as plsc

N = 10000
E = 320000
D = 128
G = 128
NUM_TASK = 10

NC = 2
NS = 16
NW = NC * NS
CHUNK = 64
NCHUNKS = E // CHUNK
TILE_CHUNKS = NCHUNKS // NW
NBUF = 4
NGRP = TILE_CHUNKS // NBUF
NLEFT = NCHUNKS - NW * TILE_CHUNKS
ROWS_PER_TILE = 632
SEG = 312
NEG = -3.4028235e38


def _edge_agg_body(h_hbm, src_hbm, dst_hbm, out_hbm,
                   src_v, dst_v, rows_v, acc_sh, *sems):
    gsems = sems[:NBUF]
    ssems = sems[NBUF:2 * NBUF]
    isem_s, isem_d = sems[2 * NBUF], sems[2 * NBUF + 1]
    c = lax.axis_index("c")
    s = lax.axis_index("s")
    wid = c * NS + s
    crow = wid * TILE_CHUNKS

    pltpu.sync_copy(src_hbm.at[pl.ds(crow, NBUF)], src_v.at[0])
    pltpu.sync_copy(dst_hbm.at[pl.ds(crow, NBUF)], dst_v.at[0])

    def _zrow(i, carry):
        for r in range(8):
            rows_v[0, i, pl.ds(r * 16, 16)] = jnp.zeros((16,), jnp.float32)
        return carry
    lax.fori_loop(0, CHUNK, _zrow, 0)
    base0 = s * ROWS_PER_TILE
    nfull = jnp.where(s < NS - 1, 9, 8)

    def _zcopy(k, carry):
        pltpu.sync_copy(rows_v.at[0],
                        acc_sh.at[pl.ds(base0 + k * CHUNK, CHUNK)])
        return carry
    lax.fori_loop(0, nfull, _zcopy, 0)

    @pl.when(s < NS - 1)
    def _():
        pltpu.sync_copy(rows_v.at[0, pl.ds(0, 56)],
                        acc_sh.at[pl.ds(base0 + 576, 56)])

    @pl.when(s == NS - 1)
    def _():
        pltpu.sync_copy(rows_v.at[0, pl.ds(0, 8)],
                        acc_sh.at[pl.ds(base0 + 512, 8)])

    plsc.subcore_barrier()

    def _group(g, carry):
        pb = lax.rem(g, 2)
        nb = 1 - pb
        nrow = crow + (g + 1) * NBUF

        @pl.when(g < NGRP - 1)
        def _():
            pltpu.async_copy(src_hbm.at[pl.ds(nrow, NBUF)], src_v.at[nb],
                             isem_s)
            pltpu.async_copy(dst_hbm.at[pl.ds(nrow, NBUF)], dst_v.at[nb],
                             isem_d)

        descs = []
        for b in range(NBUF):
            @pl.when(g > 0)
            def _(b=b):
                pltpu.make_async_copy(rows_v.at[b],
                                      acc_sh.at[dst_v.at[nb, b, 0]],
                                      ssems[b]).wait()
            descs.append(pltpu.async_copy(
                h_hbm.at[src_v.at[pb, b, 0]], rows_v.at[b], gsems[b]))
        for b in range(NBUF):
            descs[b].wait()
            pltpu.async_copy(rows_v.at[b], acc_sh.at[dst_v.at[pb, b, 0]],
                             ssems[b], add=True)

        @pl.when(g < NGRP - 1)
        def _():
            pltpu.make_async_copy(src_hbm.at[pl.ds(nrow, NBUF)],
                                  src_v.at[nb], isem_s).wait()
            pltpu.make_async_copy(dst_hbm.at[pl.ds(nrow, NBUF)],
                                  dst_v.at[nb], isem_d).wait()
        return carry
    lax.fori_loop(0, NGRP, _group, 0)

    last_pb = (NGRP - 1) % 2
    for b in range(NBUF):
        pltpu.make_async_copy(rows_v.at[b],
                              acc_sh.at[dst_v.at[last_pb, b, 0]],
                              ssems[b]).wait()

    @pl.when(wid < NLEFT)
    def _():
        lrow = NW * TILE_CHUNKS + wid
        pltpu.sync_copy(src_hbm.at[pl.ds(lrow, 1)], src_v.at[0, pl.ds(0, 1)])
        pltpu.sync_copy(dst_hbm.at[pl.ds(lrow, 1)], dst_v.at[0, pl.ds(0, 1)])
        pltpu.async_copy(h_hbm.at[src_v.at[0, 0, 0]], rows_v.at[0],
                         gsems[0]).wait()
        pltpu.sync_copy(rows_v.at[0], acc_sh.at[dst_v.at[0, 0, 0]], add=True)

    plsc.subcore_barrier()

    @pl.when(s < NS - 1)
    def _():
        pltpu.sync_copy(acc_sh.at[pl.ds(base0, ROWS_PER_TILE)],
                        out_hbm.at[c, pl.ds(base0, ROWS_PER_TILE)])

    @pl.when(s == NS - 1)
    def _():
        pltpu.sync_copy(acc_sh.at[pl.ds(base0, 520)],
                        out_hbm.at[c, pl.ds(base0, 520)])


@functools.cache
def _edge_agg_kernel():
    mesh = plsc.VectorSubcoreMesh(core_axis_name="c", subcore_axis_name="s")
    return pl.kernel(
        _edge_agg_body,
        out_type=jax.ShapeDtypeStruct((NC, N, D), jnp.float32),
        mesh=mesh,
        scratch_types=[
            pltpu.VMEM((2, NBUF, 1, CHUNK), jnp.int32),
            pltpu.VMEM((2, NBUF, 1, CHUNK), jnp.int32),
            pltpu.VMEM((NBUF, CHUNK, D), jnp.float32),
            pltpu.VMEM_SHARED((N, D), jnp.float32),
        ] + [pltpu.SemaphoreType.DMA] * (2 * NBUF + 2),
    )


def _edge_agg(h, src, dst):
    src2 = src.reshape(NCHUNKS, 1, CHUNK)
    dst2 = dst.reshape(NCHUNKS, 1, CHUNK)
    return _edge_agg_kernel()(h, src2, dst2)


def _segmax_body(h_hbm, batch_hbm, out_hbm, batch_v, rows_v, acc_v):
    c = lax.axis_index("c")
    s = lax.axis_index("s")
    wid = c * NS + s
    base = wid * SEG

    pltpu.sync_copy(batch_hbm.at[pl.ds(base, SEG)], batch_v.at[pl.ds(0, SEG)])
    pltpu.sync_copy(h_hbm.at[pl.ds(base, SEG)], rows_v.at[pl.ds(0, SEG)])

    @pl.when(wid == NW - 1)
    def _():
        pltpu.sync_copy(batch_hbm.at[pl.ds(N - 16, 16)],
                        batch_v.at[pl.ds(SEG, 16)])
        pltpu.sync_copy(h_hbm.at[pl.ds(N - 16, 16)],
                        rows_v.at[pl.ds(SEG, 16)])

    neg = jnp.full((16,), NEG, jnp.float32)

    def _irow(i, carry):
        for r in range(8):
            acc_v[i, pl.ds(r * 16, 16)] = neg
        return carry
    lax.fori_loop(0, G, _irow, 0)

    n = SEG + jnp.where(wid == NW - 1, 16, 0)

    def _node(i, carry):
        g = batch_v[pl.ds(i, 16)][0]
        for r in range(8):
            sl = pl.ds(r * 16, 16)
            acc_v[g, sl] = jnp.maximum(acc_v[g, sl], rows_v[i, sl])
        return carry
    lax.fori_loop(0, n, _node, 0)

    pltpu.sync_copy(acc_v, out_hbm.at[wid])


@functools.cache
def _segmax_kernel():
    mesh = plsc.VectorSubcoreMesh(core_axis_name="c", subcore_axis_name="s")
    return pl.kernel(
        _segmax_body,
        out_type=jax.ShapeDtypeStruct((NW, G, D), jnp.float32),
        mesh=mesh,
        scratch_types=[
            pltpu.VMEM((SEG + 16 + 16,), jnp.int32),
            pltpu.VMEM((SEG + 16, D), jnp.float32),
            pltpu.VMEM((G, D), jnp.float32),
        ],
    )


def _segmax(h, batch):
    return _segmax_kernel()(h, batch)


RB = 1000
NBLK = N // RB

def _mask_dot(mask, v, pool=False):
    m = mask.astype(jnp.bfloat16)
    v1 = v.astype(jnp.bfloat16)
    v1f = v1.astype(jnp.float32)
    v2 = (v - v1f).astype(jnp.bfloat16)
    v2f = v2.astype(jnp.float32)
    v3 = (v - v1f - v2f).astype(jnp.bfloat16)
    if pool:
        dn = (((0,), (0,)), ((), ()))
        parts = [lax.dot_general(m, t, dn, preferred_element_type=jnp.float32)
                 for t in (v1, v2, v3)]
    else:
        parts = [jnp.dot(m, t, preferred_element_type=jnp.float32)
                 for t in (v1, v2, v3)]
    return parts[0] + parts[1] + parts[2]


def _bdot(a, b):
    return jnp.dot(a.astype(jnp.bfloat16), b.astype(jnp.bfloat16),
                   preferred_element_type=jnp.float32)


def _xdot(a, b):
    return jnp.dot(a, b, preferred_element_type=jnp.float32,
                   precision=lax.Precision.HIGHEST)


def _gin_body(*refs, last):
    if last:
        h_ref, hp_ref, agg_ref, b2d_ref, w1_ref, a1_ref, w2_ref, a2_ref, out_ref = refs
    else:
        (h_ref, hp_ref, agg_ref, b2d_ref, w1_ref, a1_ref, w2_ref, a2_ref,
         out_ref, pool_ref) = refs
    h = h_ref[...]
    hp = hp_ref[...]
    z0 = a2_ref[3:4, :] * h + agg_ref[0] + agg_ref[1]
    t = _bdot(z0, w1_ref[...])
    t = a1_ref[1:2, :] * (t + a1_ref[0:1, :]) + a1_ref[2:3, :]
    t = jnp.maximum(t, 0.0)
    u = _bdot(t, w2_ref[...])
    u = a2_ref[1:2, :] * (u + a2_ref[0:1, :]) + a2_ref[2:3, :]
    if not last:
        u = jnp.maximum(u, 0.0)
    out_ref[...] = u + hp
    if not last:
        i = pl.program_id(0)
        mask = (b2d_ref[...] ==
                lax.broadcasted_iota(jnp.int32, (RB, G), 1)).astype(jnp.float32)
        p = _mask_dot(mask, hp, pool=True)

        @pl.when(i == 0)
        def _():
            pool_ref[...] = p

        @pl.when(i > 0)
        def _():
            pool_ref[...] += p


def _run_gin(h, h_prev, agg, batch2d, w1, a1, w2, a2, last):
    body = functools.partial(_gin_body, last=last)
    out_shape = [jax.ShapeDtypeStruct((N, D), jnp.float32)]
    out_specs = [pl.BlockSpec((RB, D), lambda i: (i, 0))]
    if not last:
        out_shape.append(jax.ShapeDtypeStruct((G, D), jnp.float32))
        out_specs.append(pl.BlockSpec((G, D), lambda i: (0, 0)))
    in_specs = [
        pl.BlockSpec((RB, D), lambda i: (i, 0)),
        pl.BlockSpec((RB, D), lambda i: (i, 0)),
        pl.BlockSpec((NC, RB, D), lambda i: (0, i, 0)),
        pl.BlockSpec((RB, 1), lambda i: (i, 0)),
        pl.BlockSpec((D, 2 * D), lambda i: (0, 0)),
        pl.BlockSpec((8, 2 * D), lambda i: (0, 0)),
        pl.BlockSpec((2 * D, D), lambda i: (0, 0)),
        pl.BlockSpec((8, D), lambda i: (0, 0)),
    ]
    return pl.pallas_call(
        body, grid=(NBLK,), in_specs=in_specs, out_specs=out_specs,
        out_shape=out_shape,
    )(h, h_prev, agg, batch2d, w1, a1, w2, a2)


def _vn_body(pool_ref, vn_ref, w1_ref, a1_ref, w2_ref, a2_ref,
             hn_ref, b2d_ref, hnext_ref, vnout_ref):
    vt = pool_ref[...] + vn_ref[...]
    t = _bdot(vt, w1_ref[...])
    t = a1_ref[1:2, :] * (t + a1_ref[0:1, :]) + a1_ref[2:3, :]
    t = jnp.maximum(t, 0.0)
    u = _bdot(t, w2_ref[...])
    u = a2_ref[1:2, :] * (u + a2_ref[0:1, :]) + a2_ref[2:3, :]
    vn_new = jnp.maximum(u, 0.0)
    vnout_ref[...] = vn_new
    mask = (b2d_ref[...] ==
            lax.broadcasted_iota(jnp.int32, (RB, G), 1)).astype(jnp.float32)
    hnext_ref[...] = hn_ref[...] + _mask_dot(mask, vn_new)


def _run_vn(pool, vn, w1, a1, w2, a2, h_new, batch2d):
    in_specs = [
        pl.BlockSpec((G, D), lambda i: (0, 0)),
        pl.BlockSpec((G, D), lambda i: (0, 0)),
        pl.BlockSpec((D, 2 * D), lambda i: (0, 0)),
        pl.BlockSpec((8, 2 * D), lambda i: (0, 0)),
        pl.BlockSpec((2 * D, D), lambda i: (0, 0)),
        pl.BlockSpec((8, D), lambda i: (0, 0)),
        pl.BlockSpec((RB, D), lambda i: (i, 0)),
        pl.BlockSpec((RB, 1), lambda i: (i, 0)),
    ]
    out_specs = [
        pl.BlockSpec((RB, D), lambda i: (i, 0)),
        pl.BlockSpec((G, D), lambda i: (0, 0)),
    ]
    out_shape = [
        jax.ShapeDtypeStruct((N, D), jnp.float32),
        jax.ShapeDtypeStruct((G, D), jnp.float32),
    ]
    return pl.pallas_call(
        _vn_body, grid=(NBLK,), in_specs=in_specs, out_specs=out_specs,
        out_shape=out_shape,
    )(pool, vn, w1, a1, w2, a2, h_new, batch2d)


def _pred_body(mx_ref, mor_ref, mac_ref, w1h_ref, w1m_ref, w1c_ref,
               b1_ref, w2_ref, b2_ref, out_ref):
    hrep = jnp.max(mx_ref[...], axis=0)
    z = (_bdot(hrep, w1h_ref[...])
         + _bdot(mor_ref[...], w1m_ref[...])
         + _bdot(mac_ref[...], w1c_ref[...])
         + b1_ref[0:1, :])
    z = jnp.maximum(z, 0.0)
    out_ref[...] = _bdot(z, w2_ref[...]) + b2_ref[0:1, :]


def _pad_rows(v, rows=8):
    v2 = v.reshape(1, -1)
    return jnp.concatenate(
        [v2, jnp.zeros((rows - 1, v2.shape[1]), jnp.float32)], axis=0)


def _aff(b, g, bb, extra=None):
    rows = [b, g, bb] + ([] if extra is None else [extra])
    m = jnp.stack(rows)
    pad = 8 - m.shape[0]
    return jnp.concatenate([m, jnp.zeros((pad, m.shape[1]), jnp.float32)], 0)


def kernel(x, edge_index, batch, morgan, maccs,
           gin_W1, gin_b1, gin_bn1_g, gin_bn1_b, gin_W2, gin_b2, gin_eps,
           bn_g, bn_b, vn_emb, vn_W1, vn_b1, vn_bn1_g, vn_bn1_b,
           vn_W2, vn_b2, vn_bn2_g, vn_bn2_b,
           pred_W1, pred_b1, pred_W2, pred_b2):
    src = edge_index[0]
    dst = edge_index[1]
    batch2d = batch.reshape(N, 1)

    vn = jnp.broadcast_to(vn_emb, (G, D))
    h_prev = x
    h = x + vn_emb[None, :]

    L = gin_W1.shape[0]
    for l in range(L):
        last = l == L - 1
        agg = _edge_agg(h, src, dst)
        a1 = _aff(gin_b1[l], gin_bn1_g[l], gin_bn1_b[l])
        a2 = _aff(gin_b2[l], bn_g[l], bn_b[l],
                  jnp.broadcast_to(1.0 + gin_eps[l], (D,)))
        if last:
            (h_new,) = _run_gin(h, h_prev, agg, batch2d,
                                gin_W1[l], a1, gin_W2[l], a2, last=True)
            h_prev = h_new
        else:
            h_new, pool = _run_gin(h, h_prev, agg, batch2d,
                                   gin_W1[l], a1, gin_W2[l], a2, last=False)
            av1 = _aff(vn_b1[l], vn_bn1_g[l], vn_bn1_b[l])
            av2 = _aff(vn_b2[l], vn_bn2_g[l], vn_bn2_b[l])
            h, vn = _run_vn(pool, vn, vn_W1[l], av1, vn_W2[l], av2,
                            h_new, batch2d)
            h_prev = h_new

    mx = _segmax(h_prev, batch)

    w1h = pred_W1[0:D]
    w1m = pred_W1[D:D + 1024]
    w1c = jnp.concatenate(
        [pred_W1[D + 1024:], jnp.zeros((256 - 167, 2 * D), jnp.float32)], 0)
    mac_p = jnp.concatenate(
        [maccs, jnp.zeros((G, 256 - 167), jnp.float32)], 1)
    b1p = _pad_rows(pred_b1)
    w2p = jnp.concatenate(
        [pred_W2, jnp.zeros((2 * D, 128 - NUM_TASK), jnp.float32)], 1)
    b2p = _pad_rows(jnp.concatenate(
        [pred_b2, jnp.zeros((128 - NUM_TASK,), jnp.float32)]))

    out = pl.pallas_call(
        _pred_body,
        out_shape=jax.ShapeDtypeStruct((G, 128), jnp.float32),
    )(mx, morgan, mac_p, w1h, w1m, w1c, b1p, w2p, b2p)
    return out[:, :NUM_TASK]
